# Initial kernel scaffold; baseline (speedup 1.0000x reference)
import jax
import jax.numpy as jnp
from jax.experimental import pallas as pl


def kernel(input, vertical, horizontal, offset_x, offset_y, mask):
    raise NotImplementedError("write your pallas kernel here")


# speedup vs baseline: 413.7457x; 413.7457x over previous
import functools

import jax
import jax.numpy as jnp
from jax import lax
from jax.experimental import pallas as pl
from jax.experimental.pallas import tpu as pltpu
from jax.experimental.pallas import tpu_sc as plsc

B, C, F, H, W = 8, 3, 5, 224, 224
HIN, WIN = H + F - 1, W + F - 1
NC, NS, L = 2, 16, 16
RB = 28
NQ = H // RB
NTASK = B * C * NQ
TPW = NTASK // (NC * NS)
GX = W // L


def _body(in_ref, vert_ref, horiz_ref, offx_ref, offy_ref, mask_ref,
          out_ref, img, acc, offx_v, offy_v, msk_v, vert_v, horiz_v):
  wid = lax.axis_index("s") * NC + lax.axis_index("c")
  xiota = lax.broadcasted_iota(jnp.int32, (L,), 0).astype(jnp.float32)
  fzero = jnp.zeros((L,), jnp.float32)

  def task_loop(ti, carry):
    t = wid * TPW + ti
    bc = t // NQ
    q = t % NQ
    b = bc // C
    c = bc % C
    r0 = q * RB

    pltpu.sync_copy(in_ref.at[b, c], img)
    pltpu.sync_copy(horiz_ref.at[b, :, pl.ds(r0, RB), :], horiz_v)

    def zero_r(r, _):
      def zero_g(g, _):
        acc[r, pl.ds(g * L, L)] = fzero
        return 0
      return lax.fori_loop(0, GX, zero_g, 0)
    lax.fori_loop(0, RB, zero_r, 0)

    def fy_loop(fy, _):
      pltpu.sync_copy(vert_ref.at[b, fy, pl.ds(r0, RB), :], vert_v)

      def fx_loop(fx, _):
        k = fy * F + fx
        pltpu.sync_copy(offx_ref.at[b, k, pl.ds(r0, RB), :], offx_v)
        pltpu.sync_copy(offy_ref.at[b, k, pl.ds(r0, RB), :], offy_v)
        pltpu.sync_copy(mask_ref.at[b, k, pl.ds(r0, RB), :], msk_v)

        def r_loop(r, _):
          ybase = (r0 + r + fy).astype(jnp.float32)

          def g_loop(g, _):
            sl = pl.ds(g * L, L)
            posy = offy_v[r, sl] + ybase
            posx = offx_v[r, sl] + ((g * L + fx).astype(jnp.float32) + xiota)
            posy = jnp.clip(posy, -2.0, float(HIN + 2))
            posx = jnp.clip(posx, -2.0, float(WIN + 2))
            ty = posy.astype(jnp.int32)
            tx = posx.astype(jnp.int32)
            y0 = jnp.where(ty.astype(jnp.float32) > posy, ty - 1, ty)
            x0 = jnp.where(tx.astype(jnp.float32) > posx, tx - 1, tx)
            ay = posy - y0.astype(jnp.float32)
            ax = posx - x0.astype(jnp.float32)
            y1 = y0 + 1
            x1 = x0 + 1
            vy0 = (y0 >= 0) & (y0 < HIN)
            vy1 = (y1 >= 0) & (y1 < HIN)
            vx0 = (x0 >= 0) & (x0 < WIN)
            vx1 = (x1 >= 0) & (x1 < WIN)
            yc0 = jnp.clip(y0, 0, HIN - 1)
            yc1 = jnp.clip(y1, 0, HIN - 1)
            xc0 = jnp.clip(x0, 0, WIN - 1)
            xc1 = jnp.clip(x1, 0, WIN - 1)
            by = 1.0 - ay
            bx = 1.0 - ax
            w00 = jnp.where(vy0 & vx0, by * bx, 0.0)
            w01 = jnp.where(vy0 & vx1, by * ax, 0.0)
            w10 = jnp.where(vy1 & vx0, ay * bx, 0.0)
            w11 = jnp.where(vy1 & vx1, ay * ax, 0.0)
            g00 = plsc.load_gather(img, [yc0, xc0])
            g01 = plsc.load_gather(img, [yc0, xc1])
            g10 = plsc.load_gather(img, [yc1, xc0])
            g11 = plsc.load_gather(img, [yc1, xc1])
            samp = g00 * w00 + g01 * w01 + g10 * w10 + g11 * w11
            wsep = vert_v[r, sl] * horiz_v[fx, r, sl] * msk_v[r, sl]
            acc[r, sl] = acc[r, sl] + samp * wsep
            return 0
          return lax.fori_loop(0, GX, g_loop, 0)
        return lax.fori_loop(0, RB, r_loop, 0)
      return lax.fori_loop(0, F, fx_loop, 0)
    lax.fori_loop(0, F, fy_loop, 0)

    pltpu.sync_copy(acc, out_ref.at[b, c, pl.ds(r0, RB), :])
    return 0

  lax.fori_loop(0, TPW, task_loop, 0)


@jax.jit
def kernel(input, vertical, horizontal, offset_x, offset_y, mask):
  mesh = plsc.VectorSubcoreMesh(
      core_axis_name="c", subcore_axis_name="s",
      num_cores=NC, num_subcores=NS)
  f = pl.kernel(
      _body,
      out_type=jax.ShapeDtypeStruct((B, C, H, W), jnp.float32),
      mesh=mesh,
      compiler_params=pltpu.CompilerParams(
          use_tc_tiling_on_sc=False, needs_layout_passes=False),
      scratch_types=[
          pltpu.VMEM((HIN, WIN), jnp.float32),
          pltpu.VMEM((RB, W), jnp.float32),
          pltpu.VMEM((RB, W), jnp.float32),
          pltpu.VMEM((RB, W), jnp.float32),
          pltpu.VMEM((RB, W), jnp.float32),
          pltpu.VMEM((RB, W), jnp.float32),
          pltpu.VMEM((F, RB, W), jnp.float32),
      ],
  )
  return f(input, vertical, horizontal, offset_x, offset_y, mask)


# speedup vs baseline: 786.1623x; 1.9001x over previous
import jax
import jax.numpy as jnp
from jax import lax
from jax.experimental import pallas as pl
from jax.experimental.pallas import tpu as pltpu
from jax.experimental.pallas import tpu_sc as plsc

B, C, F, H, W = 8, 3, 5, 224, 224
HIN, WIN = H + F - 1, W + F - 1
NC, NS, L = 2, 16, 16
RB = 28
NQ = H // RB
NTASK = B * C * NQ
TPW = NTASK // (NC * NS)
GX = W // L

PADW = 240
PADH = 234
PADN = PADH * PADW
PX, PY = 8, 2
IDXC = (PY - 4) * PADW + (PX - 4)
BLO, BHI = 2.0, 232.9


def _body(in_ref, vert_ref, horiz_ref, offx_ref, offy_ref, mask_ref,
          out_ref, img, acc, offx_v, offy_v, msk_v, vert_v, horiz_v):
  wid = lax.axis_index("s") * NC + lax.axis_index("c")
  xiota = lax.broadcasted_iota(jnp.int32, (L,), 0).astype(jnp.float32)
  fzero = jnp.zeros((L,), jnp.float32)

  def task_loop(ti, carry):
    t = wid * TPW + ti
    bc = t // NQ
    q = t % NQ
    b = bc // C
    c = bc % C
    r0 = q * RB

    pltpu.sync_copy(in_ref.at[b, c], img)
    pltpu.sync_copy(horiz_ref.at[b, :, pl.ds(r0, RB), :], horiz_v)

    def zero_r(r, _):
      def zero_g(g, _):
        acc[r, pl.ds(g * L, L)] = fzero
        return 0
      return lax.fori_loop(0, GX, zero_g, 0)
    lax.fori_loop(0, RB, zero_r, 0)

    def fy_loop(fy, _):
      pltpu.sync_copy(vert_ref.at[b, fy, pl.ds(r0, RB), :], vert_v)

      def fx_loop(fx, _):
        k = fy * F + fx
        pltpu.sync_copy(offx_ref.at[b, k, pl.ds(r0, RB), :], offx_v)
        pltpu.sync_copy(offy_ref.at[b, k, pl.ds(r0, RB), :], offy_v)
        pltpu.sync_copy(mask_ref.at[b, k, pl.ds(r0, RB), :], msk_v)
        fxf = (fx + 4).astype(jnp.float32)

        def r_loop(r, _):
          ybase = (r0 + r + fy + 4).astype(jnp.float32)

          @plsc.parallel_loop(0, GX, 1, unroll=2)
          def g_loop(g):
            sl = pl.ds(g * L, L)
            posy = offy_v[r, sl] + ybase
            posx = (offx_v[r, sl] + ((g * L).astype(jnp.float32) + fxf)) + xiota
            posy = jnp.clip(posy, BLO, BHI)
            posx = jnp.clip(posx, BLO, BHI)
            ty = posy.astype(jnp.int32)
            tx = posx.astype(jnp.int32)
            ay = posy - ty.astype(jnp.float32)
            ax = posx - tx.astype(jnp.float32)
            f00 = ty * PADW + (tx + IDXC)
            f10 = f00 + PADW
            g00 = plsc.load_gather(img, [f00])
            g01 = plsc.load_gather(img, [f00 + 1])
            g10 = plsc.load_gather(img, [f10])
            g11 = plsc.load_gather(img, [f10 + 1])
            by = 1.0 - ay
            bx = 1.0 - ax
            samp = (g00 * (by * bx) + g01 * (by * ax)
                    + g10 * (ay * bx) + g11 * (ay * ax))
            wsep = vert_v[r, sl] * horiz_v[fx, r, sl] * msk_v[r, sl]
            acc[r, sl] = acc[r, sl] + samp * wsep

          return 0
        return lax.fori_loop(0, RB, r_loop, 0)
      return lax.fori_loop(0, F, fx_loop, 0)
    lax.fori_loop(0, F, fy_loop, 0)

    pltpu.sync_copy(acc, out_ref.at[b, c, pl.ds(r0, RB), :])
    return 0

  lax.fori_loop(0, TPW, task_loop, 0)


@jax.jit
def kernel(input, vertical, horizontal, offset_x, offset_y, mask):
  inp = jnp.pad(input, ((0, 0), (0, 0),
                        (PY, PADH - HIN - PY),
                        (PX, PADW - WIN - PX))).reshape(B, C, PADN)
  mesh = plsc.VectorSubcoreMesh(
      core_axis_name="c", subcore_axis_name="s",
      num_cores=NC, num_subcores=NS)
  f = pl.kernel(
      _body,
      out_type=jax.ShapeDtypeStruct((B, C, H, W), jnp.float32),
      mesh=mesh,
      compiler_params=pltpu.CompilerParams(
          use_tc_tiling_on_sc=False, needs_layout_passes=False),
      scratch_types=[
          pltpu.VMEM((PADN,), jnp.float32),
          pltpu.VMEM((RB, W), jnp.float32),
          pltpu.VMEM((RB, W), jnp.float32),
          pltpu.VMEM((RB, W), jnp.float32),
          pltpu.VMEM((RB, W), jnp.float32),
          pltpu.VMEM((RB, W), jnp.float32),
          pltpu.VMEM((F, RB, W), jnp.float32),
      ],
  )
  return f(inp, vertical, horizontal, offset_x, offset_y, mask)


# speedup vs baseline: 1170.8383x; 1.4893x over previous
import jax
import jax.numpy as jnp
from jax import lax
from jax.experimental import pallas as pl
from jax.experimental.pallas import tpu as pltpu
from jax.experimental.pallas import tpu_sc as plsc

B, C, F, H, W = 8, 3, 5, 224, 224
HIN, WIN = H + F - 1, W + F - 1
NC, NS, L = 2, 16, 16
RB = 28
NQ = H // RB
NTASK = B * C * NQ
TPW = NTASK // (NC * NS)
GX = W // L
K = F * F

PADW = 240
PADH = 234
PADN = PADH * PADW
PX, PY = 8, 2
IDXC = (PY - 4) * PADW + (PX - 4)
BLO, BHI = 2.0, 232.9


def _body(in_ref, vert_ref, horiz_ref, offx_ref, offy_ref, mask_ref,
          out_ref, img, acc,
          offx_a, offy_a, msk_a, vert_a, horiz_a,
          offx_b, offy_b, msk_b, vert_b, horiz_b,
          sem_a, sem_b):
  wid = lax.axis_index("s") * NC + lax.axis_index("c")
  xiota = lax.broadcasted_iota(jnp.int32, (L,), 0).astype(jnp.float32)
  fzero = jnp.zeros((L,), jnp.float32)

  bufs = ((offx_a, offy_a, msk_a, vert_a, horiz_a, sem_a),
          (offx_b, offy_b, msk_b, vert_b, horiz_b, sem_b))

  def issue(k, b, r0, par):
    k = jnp.minimum(k, K - 1)
    fy = k // F
    fx = k % F
    ox, oy, mk, vt, hz, sem = bufs[par]
    pltpu.async_copy(offx_ref.at[b, k, pl.ds(r0, RB), :], ox, sem)
    pltpu.async_copy(offy_ref.at[b, k, pl.ds(r0, RB), :], oy, sem)
    pltpu.async_copy(mask_ref.at[b, k, pl.ds(r0, RB), :], mk, sem)
    pltpu.async_copy(vert_ref.at[b, fy, pl.ds(r0, RB), :], vt, sem)
    pltpu.async_copy(horiz_ref.at[b, fx, pl.ds(r0, RB), :], hz, sem)

  def drain(b, r0, par):
    ox, oy, mk, vt, hz, sem = bufs[par]
    pltpu.make_async_copy(offx_ref.at[0, 0, pl.ds(0, RB), :], ox, sem).wait()
    pltpu.make_async_copy(offy_ref.at[0, 0, pl.ds(0, RB), :], oy, sem).wait()
    pltpu.make_async_copy(mask_ref.at[0, 0, pl.ds(0, RB), :], mk, sem).wait()
    pltpu.make_async_copy(vert_ref.at[0, 0, pl.ds(0, RB), :], vt, sem).wait()
    pltpu.make_async_copy(horiz_ref.at[0, 0, pl.ds(0, RB), :], hz, sem).wait()

  def compute(k, r0, par):
    ox, oy, mk, vt, hz, _ = bufs[par]
    fy = k // F
    fxf = (k % F + 4).astype(jnp.float32)

    def r_loop(r, _):
      ybase = (r0 + r + fy + 4).astype(jnp.float32)

      @plsc.parallel_loop(0, GX, 1, unroll=2)
      def g_loop(g):
        sl = pl.ds(g * L, L)
        posy = oy[r, sl] + ybase
        posx = (ox[r, sl] + ((g * L).astype(jnp.float32) + fxf)) + xiota
        posy = jnp.clip(posy, BLO, BHI)
        posx = jnp.clip(posx, BLO, BHI)
        ty = posy.astype(jnp.int32)
        tx = posx.astype(jnp.int32)
        ay = posy - ty.astype(jnp.float32)
        ax = posx - tx.astype(jnp.float32)
        f00 = ty * PADW + (tx + IDXC)
        f10 = f00 + PADW
        g00 = plsc.load_gather(img, [f00])
        g01 = plsc.load_gather(img, [f00 + 1])
        g10 = plsc.load_gather(img, [f10])
        g11 = plsc.load_gather(img, [f10 + 1])
        by = 1.0 - ay
        bx = 1.0 - ax
        samp = by * (bx * g00 + ax * g01) + ay * (bx * g10 + ax * g11)
        wsep = vt[r, sl] * hz[r, sl] * mk[r, sl]
        acc[r, sl] = acc[r, sl] + samp * wsep

      return 0
    lax.fori_loop(0, RB, r_loop, 0)

  def task_loop(ti, carry):
    t = wid * TPW + ti
    bc = t // NQ
    q = t % NQ
    b = bc // C
    c = bc % C
    r0 = q * RB

    pltpu.sync_copy(in_ref.at[b, c], img)
    issue(jnp.int32(0), b, r0, 0)
    issue(jnp.int32(1), b, r0, 1)

    def zero_r(r, _):
      def zero_g(g, _):
        acc[r, pl.ds(g * L, L)] = fzero
        return 0
      return lax.fori_loop(0, GX, zero_g, 0)
    lax.fori_loop(0, RB, zero_r, 0)

    def kk_loop(kk, _):
      k = 2 * kk
      drain(b, r0, 0)
      compute(k, r0, 0)
      issue(k + 2, b, r0, 0)
      drain(b, r0, 1)
      compute(k + 1, r0, 1)
      issue(k + 3, b, r0, 1)
      return 0
    lax.fori_loop(0, (K - 1) // 2, kk_loop, 0)

    drain(b, r0, 0)
    compute(jnp.int32(K - 1), r0, 0)
    drain(b, r0, 1)

    pltpu.sync_copy(acc, out_ref.at[b, c, pl.ds(r0, RB), :])
    return 0

  lax.fori_loop(0, TPW, task_loop, 0)


@jax.jit
def kernel(input, vertical, horizontal, offset_x, offset_y, mask):
  inp = jnp.pad(input, ((0, 0), (0, 0),
                        (PY, PADH - HIN - PY),
                        (PX, PADW - WIN - PX))).reshape(B, C, PADN)
  mesh = plsc.VectorSubcoreMesh(
      core_axis_name="c", subcore_axis_name="s",
      num_cores=NC, num_subcores=NS)
  blk = pltpu.VMEM((RB, W), jnp.float32)
  f = pl.kernel(
      _body,
      out_type=jax.ShapeDtypeStruct((B, C, H, W), jnp.float32),
      mesh=mesh,
      compiler_params=pltpu.CompilerParams(
          use_tc_tiling_on_sc=False, needs_layout_passes=False),
      scratch_types=[
          pltpu.VMEM((PADN,), jnp.float32),
          blk,
          blk, blk, blk, blk, blk,
          blk, blk, blk, blk, blk,
          pltpu.SemaphoreType.DMA,
          pltpu.SemaphoreType.DMA,
      ],
  )
  return f(inp, vertical, horizontal, offset_x, offset_y, mask)


# speedup vs baseline: 1275.0388x; 1.0890x over previous
import jax
import jax.numpy as jnp
from jax import lax
from jax.experimental import pallas as pl
from jax.experimental.pallas import tpu as pltpu
from jax.experimental.pallas import tpu_sc as plsc

B, C, F, H, W = 8, 3, 5, 224, 224
HIN, WIN = H + F - 1, W + F - 1
NC, NS, L = 2, 16, 16
RB = 28
RBW = RB * W
NQ = H // RB
NTASK = B * C * NQ
TPW = NTASK // (NC * NS)
GX = W // L
NG = RB * GX
K = F * F

PADW = 240
PADH = 234
PADN = PADH * PADW
PX, PY = 8, 2
IDXC = (PY - 4) * PADW + (PX - 4)
BLO, BHI = 2.0, 232.9


def _body(in_ref, vert_ref, horiz_ref, offx_ref, offy_ref, mask_ref,
          out_ref, img, acc,
          offx_a, offy_a, msk_a, vert_a, horiz_a,
          offx_b, offy_b, msk_b, vert_b, horiz_b,
          sem_a, sem_b):
  wid = lax.axis_index("s") * NC + lax.axis_index("c")
  xiota = lax.broadcasted_iota(jnp.int32, (L,), 0).astype(jnp.float32)
  fzero = jnp.zeros((L,), jnp.float32)

  bufs = ((offx_a, offy_a, msk_a, vert_a, horiz_a, sem_a),
          (offx_b, offy_b, msk_b, vert_b, horiz_b, sem_b))

  def issue(k, b, r0w, par):
    k = jnp.minimum(k, K - 1)
    fy = k // F
    fx = k % F
    ox, oy, mk, vt, hz, sem = bufs[par]
    pltpu.async_copy(offx_ref.at[b, k, pl.ds(r0w, RBW)], ox, sem)
    pltpu.async_copy(offy_ref.at[b, k, pl.ds(r0w, RBW)], oy, sem)
    pltpu.async_copy(mask_ref.at[b, k, pl.ds(r0w, RBW)], mk, sem)
    pltpu.async_copy(vert_ref.at[b, fy, pl.ds(r0w, RBW)], vt, sem)
    pltpu.async_copy(horiz_ref.at[b, fx, pl.ds(r0w, RBW)], hz, sem)

  def drain(par):
    ox, oy, mk, vt, hz, sem = bufs[par]
    pltpu.make_async_copy(offx_ref.at[0, 0, pl.ds(0, RBW)], ox, sem).wait()
    pltpu.make_async_copy(offy_ref.at[0, 0, pl.ds(0, RBW)], oy, sem).wait()
    pltpu.make_async_copy(mask_ref.at[0, 0, pl.ds(0, RBW)], mk, sem).wait()
    pltpu.make_async_copy(vert_ref.at[0, 0, pl.ds(0, RBW)], vt, sem).wait()
    pltpu.make_async_copy(horiz_ref.at[0, 0, pl.ds(0, RBW)], hz, sem).wait()

  def compute(k, r0, par):
    ox, oy, mk, vt, hz, _ = bufs[par]
    fy = k // F
    ybase0 = r0 + fy + 4
    fxf = (k % F + 4).astype(jnp.float32)

    @plsc.parallel_loop(0, NG, 1, unroll=2)
    def g_loop(i):
      sl = pl.ds(i * L, L)
      ybase = (ybase0 + i // GX).astype(jnp.float32)
      xb = ((i % GX) * L).astype(jnp.float32) + fxf
      posy = oy[sl] + ybase
      posx = (ox[sl] + xb) + xiota
      posy = jnp.clip(posy, BLO, BHI)
      posx = jnp.clip(posx, BLO, BHI)
      ty = posy.astype(jnp.int32)
      tx = posx.astype(jnp.int32)
      ay = posy - ty.astype(jnp.float32)
      ax = posx - tx.astype(jnp.float32)
      f00 = ty * PADW + (tx + IDXC)
      f10 = f00 + PADW
      g00 = plsc.load_gather(img, [f00])
      g01 = plsc.load_gather(img, [f00 + 1])
      g10 = plsc.load_gather(img, [f10])
      g11 = plsc.load_gather(img, [f10 + 1])
      by = 1.0 - ay
      bx = 1.0 - ax
      samp = by * (bx * g00 + ax * g01) + ay * (bx * g10 + ax * g11)
      wsep = vt[sl] * hz[sl] * mk[sl]
      acc[sl] = acc[sl] + samp * wsep

  def task_loop(ti, carry):
    t = wid * TPW + ti
    bc = t // NQ
    q = t % NQ
    b = bc // C
    c = bc % C
    r0 = q * RB
    r0w = q * RBW

    pltpu.sync_copy(in_ref.at[b, c], img)
    issue(jnp.int32(0), b, r0w, 0)
    issue(jnp.int32(1), b, r0w, 1)

    def zero_g(i, _):
      acc[pl.ds(i * L, L)] = fzero
      return 0
    lax.fori_loop(0, NG, zero_g, 0)

    def kk_loop(kk, _):
      k = 2 * kk
      drain(0)
      compute(k, r0, 0)
      issue(k + 2, b, r0w, 0)
      drain(1)
      compute(k + 1, r0, 1)
      issue(k + 3, b, r0w, 1)
      return 0
    lax.fori_loop(0, (K - 1) // 2, kk_loop, 0)

    drain(0)
    compute(jnp.int32(K - 1), r0, 0)
    drain(1)

    pltpu.sync_copy(acc, out_ref.at[b, c, pl.ds(r0w, RBW)])
    return 0

  lax.fori_loop(0, TPW, task_loop, 0)


@jax.jit
def kernel(input, vertical, horizontal, offset_x, offset_y, mask):
  inp = jnp.pad(input, ((0, 0), (0, 0),
                        (PY, PADH - HIN - PY),
                        (PX, PADW - WIN - PX))).reshape(B, C, PADN)
  mesh = plsc.VectorSubcoreMesh(
      core_axis_name="c", subcore_axis_name="s",
      num_cores=NC, num_subcores=NS)
  blk = pltpu.VMEM((RBW,), jnp.float32)
  f = pl.kernel(
      _body,
      out_type=jax.ShapeDtypeStruct((B, C, H * W), jnp.float32),
      mesh=mesh,
      compiler_params=pltpu.CompilerParams(
          use_tc_tiling_on_sc=False, needs_layout_passes=False),
      scratch_types=[
          pltpu.VMEM((PADN,), jnp.float32),
          blk,
          blk, blk, blk, blk, blk,
          blk, blk, blk, blk, blk,
          pltpu.SemaphoreType.DMA,
          pltpu.SemaphoreType.DMA,
      ],
  )
  out = f(inp, vertical.reshape(B, F, H * W), horizontal.reshape(B, F, H * W),
          offset_x.reshape(B, K, H * W), offset_y.reshape(B, K, H * W),
          mask.reshape(B, K, H * W))
  return out.reshape(B, C, H, W)
